# Initial kernel scaffold; baseline (speedup 1.0000x reference)
import jax
import jax.numpy as jnp
from jax.experimental import pallas as pl


def kernel(embed, wtap, w1, b1, w2, b2, x_ids):
    raise NotImplementedError("write your pallas kernel here")


# speedup vs baseline: 1.0332x; 1.0332x over previous
import functools

import jax
import jax.numpy as jnp
from jax.experimental import pallas as pl
from jax.experimental.pallas import tpu as pltpu

_NEG_INF = -1e30
_KERNEL_WINS = (3, 4, 5)
_DIM_CHANNEL = 256
_PAD_ID = 0
_NUM_CLASS = 20


def _round_up(x, m):
    return ((x + m - 1) // m) * m


def _fused_kernel(mpad_ref, emb_ref, wtap_ref, w1_ref, b1_ref, w2_ref, b2_ref,
                  out_ref, *, kernel_wins, dim_channel):
    Bt, L = mpad_ref.shape
    _, L_ext, E = emb_ref.shape
    n_taps = L_ext - L + 1
    C = dim_channel

    emb = emb_ref[...]
    unf = jnp.concatenate([emb[:, k:k + L, :] for k in range(n_taps)],
                          axis=-1).reshape(Bt * L, n_taps * E)

    mpad = mpad_ref[...]
    pooled = []
    for i, win in enumerate(kernel_wins):
        depth = win * E
        a = jnp.dot(unf[:, :depth], wtap_ref[:depth, i * C:(i + 1) * C],
                    preferred_element_type=jnp.float32)
        a = a.reshape(Bt, L, C)
        if win > 1:
            m = jnp.concatenate(
                [mpad[:, win - 1:],
                 jnp.full((Bt, win - 1), _NEG_INF, jnp.float32)], axis=1)
        else:
            m = mpad
        pooled.append(jnp.max(a + m[:, :, None], axis=1))
    pooled = jnp.concatenate(pooled, axis=-1)

    h = jnp.dot(pooled, w1_ref[...], preferred_element_type=jnp.float32)
    h = jnp.maximum(h + b1_ref[...], 0.0)
    out_ref[...] = jnp.dot(h, w2_ref[...],
                           preferred_element_type=jnp.float32) + b2_ref[...]


@jax.jit
def _forward(embed, wtap, w1, b1, w2, b2, x_ids):
    B, L = x_ids.shape
    E = embed.shape[1]
    KP, CP = wtap.shape
    n_taps = KP // E
    FP = w1.shape[1]
    NCP = w2.shape[1]
    L_ext = L + n_taps - 1

    Bt = 8
    B_pad = _round_up(B, Bt)
    grid = (B_pad // Bt,)

    x_ext = jnp.pad(x_ids, ((0, B_pad - B), (0, n_taps - 1)),
                    constant_values=_PAD_ID)
    emb = embed[x_ext]
    mpad = jnp.where(x_ext[:, :L] == _PAD_ID,
                     jnp.float32(-100.0), jnp.float32(0.0))

    kern = functools.partial(_fused_kernel, kernel_wins=_KERNEL_WINS,
                             dim_channel=_DIM_CHANNEL)
    out = pl.pallas_call(
        kern,
        out_shape=jax.ShapeDtypeStruct((B_pad, NCP), jnp.float32),
        grid=grid,
        in_specs=[
            pl.BlockSpec((Bt, L), lambda b: (b, 0)),
            pl.BlockSpec((Bt, L_ext, E), lambda b: (b, 0, 0)),
            pl.BlockSpec((KP, CP), lambda b: (0, 0)),
            pl.BlockSpec((CP, FP), lambda b: (0, 0)),
            pl.BlockSpec((1, FP), lambda b: (0, 0)),
            pl.BlockSpec((FP, NCP), lambda b: (0, 0)),
            pl.BlockSpec((1, NCP), lambda b: (0, 0)),
        ],
        out_specs=pl.BlockSpec((Bt, NCP), lambda b: (b, 0)),
        compiler_params=pltpu.CompilerParams(
            dimension_semantics=("parallel",),
            vmem_limit_bytes=64 * 1024 * 1024),
    )(mpad, emb, wtap, w1, b1, w2, b2)

    return out[:B, :_NUM_CLASS]


def kernel(embed, wtap, w1, b1, w2, b2, x_ids):
    return _forward(embed, wtap, w1, b1, w2, b2, x_ids)


# speedup vs baseline: 1.1374x; 1.1009x over previous
import functools

import jax
import jax.numpy as jnp
from jax.experimental import pallas as pl
from jax.experimental.pallas import tpu as pltpu

_NEG_INF = -1e30
_KERNEL_WINS = (3, 4, 5)
_DIM_CHANNEL = 256
_PAD_ID = 0
_NUM_CLASS = 20


def _round_up(x, m):
    return ((x + m - 1) // m) * m


def _fused_kernel(xb_ref, xsh_ref, xcol_ref, mpad_ref, tbl_ref, wtap_ref,
                  w1_ref, b1_ref, w2_ref, b2_ref, out_ref, gbuf,
                  *, kernel_wins, dim_channel, bt, l_seq, l_ext):
    Bt, L, L_ext = bt, l_seq, l_ext
    E = tbl_ref.shape[1]
    M = Bt * L_ext
    C = dim_channel

    submask = [jax.lax.broadcasted_iota(jnp.int32, (8, E), 0) == i
               for i in range(8)]

    def gather_chunk(c, _):
        acc = None
        for i in range(8):
            base = xb_ref[0, 0, c * 8 + i]
            sh = xsh_ref[0, 0, c * 8 + i]
            chunk = tbl_ref[pl.ds(pl.multiple_of(base, 16), 16), :]
            ci = pltpu.bitcast(chunk, jnp.int32)
            rot = pltpu.roll(ci, sh, axis=0)
            acc = rot if acc is None else jnp.where(submask[i], rot, acc)
        gbuf[pl.ds(pl.multiple_of(c * 8, 8), 8), :] = acc
        return 0

    jax.lax.fori_loop(0, M // 8, gather_chunk, 0)

    odd = (xcol_ref[...] & 1) == 1
    g = gbuf[...]
    bits = jnp.where(odd, g & jnp.int32(-65536), g << 16)
    emb = pltpu.bitcast(bits, jnp.float32).astype(jnp.bfloat16)
    emb = emb.reshape(Bt, L_ext, E)

    n_taps = L_ext - L + 1
    unf = jnp.concatenate([emb[:, k:k + L, :] for k in range(n_taps)],
                          axis=-1).reshape(Bt * L, n_taps * E)

    mpad = mpad_ref[...]
    pooled = []
    for i, win in enumerate(kernel_wins):
        depth = win * E
        a = jnp.dot(unf[:, :depth], wtap_ref[:depth, i * C:(i + 1) * C],
                    preferred_element_type=jnp.float32)
        a = a.reshape(Bt, L, C)
        if win > 1:
            m = jnp.concatenate(
                [mpad[:, win - 1:],
                 jnp.full((Bt, win - 1), _NEG_INF, jnp.float32)], axis=1)
        else:
            m = mpad
        pooled.append(jnp.max(a + m[:, :, None], axis=1))
    pooled = jnp.concatenate(pooled, axis=-1)

    h = jnp.dot(pooled, w1_ref[...], preferred_element_type=jnp.float32)
    h = jnp.maximum(h + b1_ref[...], 0.0)
    out_ref[...] = jnp.dot(h, w2_ref[...],
                           preferred_element_type=jnp.float32) + b2_ref[...]


@jax.jit
def _forward(embed, wtap, w1, b1, w2, b2, x_ids):
    B, L = x_ids.shape
    E = embed.shape[1]
    KP, CP = wtap.shape
    n_taps = KP // E
    FP = w1.shape[1]
    NCP = w2.shape[1]
    L_ext = L + n_taps - 1

    Bt = 8
    B_pad = _round_up(B, Bt)
    NB = B_pad // Bt
    M = Bt * L_ext
    grid = (NB,)

    x_ext = jnp.pad(x_ids, ((0, B_pad - B), (0, n_taps - 1)),
                    constant_values=_PAD_ID)
    xf = x_ext.reshape(-1)
    xbase = ((xf >> 4) << 4).reshape(NB, 1, M)
    slot = jnp.arange(B_pad * L_ext, dtype=jnp.int32) & 7
    xshift = ((slot - ((xf >> 1) & 7)) & 7).reshape(NB, 1, M)
    xcol = xf.reshape(B_pad * L_ext, 1)
    mpad = jnp.where(x_ext[:, :L] == _PAD_ID,
                     jnp.float32(-100.0), jnp.float32(0.0))

    kern = functools.partial(_fused_kernel, kernel_wins=_KERNEL_WINS,
                             dim_channel=_DIM_CHANNEL, bt=Bt, l_seq=L,
                             l_ext=L_ext)
    out = pl.pallas_call(
        kern,
        out_shape=jax.ShapeDtypeStruct((B_pad, NCP), jnp.float32),
        grid=grid,
        in_specs=[
            pl.BlockSpec((1, 1, M), lambda b: (b, 0, 0),
                         memory_space=pltpu.SMEM),
            pl.BlockSpec((1, 1, M), lambda b: (b, 0, 0),
                         memory_space=pltpu.SMEM),
            pl.BlockSpec((M, 1), lambda b: (b, 0)),
            pl.BlockSpec((Bt, L), lambda b: (b, 0)),
            pl.BlockSpec(embed.shape, lambda b: (0, 0)),
            pl.BlockSpec((KP, CP), lambda b: (0, 0)),
            pl.BlockSpec((CP, FP), lambda b: (0, 0)),
            pl.BlockSpec((1, FP), lambda b: (0, 0)),
            pl.BlockSpec((FP, NCP), lambda b: (0, 0)),
            pl.BlockSpec((1, NCP), lambda b: (0, 0)),
        ],
        out_specs=pl.BlockSpec((Bt, NCP), lambda b: (b, 0)),
        scratch_shapes=[pltpu.VMEM((M, E), jnp.int32)],
        compiler_params=pltpu.CompilerParams(
            dimension_semantics=("parallel",),
            vmem_limit_bytes=60 * 1024 * 1024),
    )(xbase, xshift, xcol, mpad, embed, wtap, w1, b1, w2, b2)

    return out[:B, :_NUM_CLASS]


def kernel(embed, wtap, w1, b1, w2, b2, x_ids):
    return _forward(embed, wtap, w1, b1, w2, b2, x_ids)


# speedup vs baseline: 1.2752x; 1.1211x over previous
import functools

import jax
import jax.numpy as jnp
from jax.experimental import pallas as pl
from jax.experimental.pallas import tpu as pltpu

_NEG_INF = -1e30
_KERNEL_WINS = (3, 4, 5)
_DIM_CHANNEL = 256
_PAD_ID = 0
_NUM_CLASS = 20


def _round_up(x, m):
    return ((x + m - 1) // m) * m


def _fused_kernel(xb_ref, xsh_ref, xcol_ref, mpad_ref, tbl_ref, wtap_ref,
                  w1_ref, b1_ref, w2_ref, b2_ref, out_ref, gbuf,
                  *, kernel_wins, dim_channel, bt, l_seq, l_ext):
    Bt, L, L_ext = bt, l_seq, l_ext
    E = tbl_ref.shape[1]
    M = Bt * L_ext
    C = dim_channel

    submask = [jax.lax.broadcasted_iota(jnp.int32, (8, E), 0) == i
               for i in range(8)]

    UN = 32

    def gather_chunk(c, _):
        rots = []
        for i in range(UN):
            base = xb_ref[0, 0, c * UN + i]
            sh = xsh_ref[0, 0, c * UN + i]
            chunk = tbl_ref[pl.ds(pl.multiple_of(base, 16), 16), :]
            ci = pltpu.bitcast(chunk, jnp.int32)
            rots.append(pltpu.roll(ci, sh, axis=0))
        for g in range(UN // 8):
            acc = rots[g * 8]
            for i in range(1, 8):
                acc = jnp.where(submask[i], rots[g * 8 + i], acc)
            gbuf[pl.ds(pl.multiple_of(c * UN + g * 8, 8), 8), :] = acc
        return 0

    jax.lax.fori_loop(0, M // UN, gather_chunk, 0)

    odd = (xcol_ref[...] & 1) == 1
    g = gbuf[...]
    bits = jnp.where(odd, g & jnp.int32(-65536), g << 16)
    emb = pltpu.bitcast(bits, jnp.float32).astype(jnp.bfloat16)
    emb = emb.reshape(Bt, L_ext, E)

    n_taps = L_ext - L + 1
    unf = jnp.concatenate([emb[:, k:k + L, :] for k in range(n_taps)],
                          axis=-1).reshape(Bt * L, n_taps * E)

    mpad = mpad_ref[...]
    pooled = []
    for i, win in enumerate(kernel_wins):
        depth = win * E
        a = jnp.dot(unf[:, :depth], wtap_ref[:depth, i * C:(i + 1) * C],
                    preferred_element_type=jnp.float32)
        a = a.reshape(Bt, L, C)
        if win > 1:
            m = jnp.concatenate(
                [mpad[:, win - 1:],
                 jnp.full((Bt, win - 1), _NEG_INF, jnp.float32)], axis=1)
        else:
            m = mpad
        pooled.append(jnp.max(a + m[:, :, None], axis=1))
    pooled = jnp.concatenate(pooled, axis=-1)

    h = jnp.dot(pooled, w1_ref[...], preferred_element_type=jnp.float32)
    h = jnp.maximum(h + b1_ref[...], 0.0)
    out_ref[...] = jnp.dot(h, w2_ref[...],
                           preferred_element_type=jnp.float32) + b2_ref[...]


@jax.jit
def _forward(embed, wtap, w1, b1, w2, b2, x_ids):
    B, L = x_ids.shape
    E = embed.shape[1]
    KP, CP = wtap.shape
    n_taps = KP // E
    FP = w1.shape[1]
    NCP = w2.shape[1]
    L_ext = L + n_taps - 1

    Bt = 8
    B_pad = _round_up(B, Bt)
    NB = B_pad // Bt
    M = Bt * L_ext
    grid = (NB,)

    x_ext = jnp.pad(x_ids, ((0, B_pad - B), (0, n_taps - 1)),
                    constant_values=_PAD_ID)
    xf = x_ext.reshape(-1)
    xbase = ((xf >> 4) << 4).reshape(NB, 1, M)
    slot = jnp.arange(B_pad * L_ext, dtype=jnp.int32) & 7
    xshift = ((slot - ((xf >> 1) & 7)) & 7).reshape(NB, 1, M)
    xcol = xf.reshape(B_pad * L_ext, 1)
    mpad = jnp.where(x_ext[:, :L] == _PAD_ID,
                     jnp.float32(-100.0), jnp.float32(0.0))

    kern = functools.partial(_fused_kernel, kernel_wins=_KERNEL_WINS,
                             dim_channel=_DIM_CHANNEL, bt=Bt, l_seq=L,
                             l_ext=L_ext)
    out = pl.pallas_call(
        kern,
        out_shape=jax.ShapeDtypeStruct((B_pad, NCP), jnp.float32),
        grid=grid,
        in_specs=[
            pl.BlockSpec((1, 1, M), lambda b: (b, 0, 0),
                         memory_space=pltpu.SMEM),
            pl.BlockSpec((1, 1, M), lambda b: (b, 0, 0),
                         memory_space=pltpu.SMEM),
            pl.BlockSpec((M, 1), lambda b: (b, 0)),
            pl.BlockSpec((Bt, L), lambda b: (b, 0)),
            pl.BlockSpec(embed.shape, lambda b: (0, 0)),
            pl.BlockSpec((KP, CP), lambda b: (0, 0)),
            pl.BlockSpec((CP, FP), lambda b: (0, 0)),
            pl.BlockSpec((1, FP), lambda b: (0, 0)),
            pl.BlockSpec((FP, NCP), lambda b: (0, 0)),
            pl.BlockSpec((1, NCP), lambda b: (0, 0)),
        ],
        out_specs=pl.BlockSpec((Bt, NCP), lambda b: (b, 0)),
        scratch_shapes=[pltpu.VMEM((M, E), jnp.int32)],
        compiler_params=pltpu.CompilerParams(
            dimension_semantics=("parallel",),
            vmem_limit_bytes=60 * 1024 * 1024),
    )(xbase, xshift, xcol, mpad, embed, wtap, w1, b1, w2, b2)

    return out[:B, :_NUM_CLASS]


def kernel(embed, wtap, w1, b1, w2, b2, x_ids):
    return _forward(embed, wtap, w1, b1, w2, b2, x_ids)
